# baseline (speedup 1.0000x reference)
import jax
import jax.numpy as jnp
from jax import lax
from jax.experimental import pallas as pl

_F = 256
_L = 32
_T = _L * _L
_D0_BLK = 8


def _body(w_ref, o_ref):
    i = pl.program_id(0)
    j = pl.program_id(1)
    w32 = w_ref[0, :_L, :]

    def sec01(div):
        k_i = lax.broadcasted_iota(jnp.int32, (_L, _T), 0)
        t_i = lax.broadcasted_iota(jnp.int32, (_L, _T), 1)
        m = ((t_i // div) % _L == k_i).astype(jnp.float32)
        b = lax.dot_general(w32, m, (((0,), (0,)), ((), ())),
                            precision=lax.Precision.HIGHEST,
                            preferred_element_type=jnp.float32)
        o_ref[...] = jnp.broadcast_to(b[:, None, :], (_F, _D0_BLK, _T))

    @pl.when(i == 0)
    def _():
        sec01(1)

    @pl.when(i == 1)
    def _():
        sec01(_L)

    @pl.when(i == 2)
    def _():
        w8 = w_ref[0, pl.ds(j * _D0_BLK, _D0_BLK), :]
        r_i = lax.broadcasted_iota(jnp.int32, (_D0_BLK, _D0_BLK), 0)
        c_i = lax.broadcasted_iota(jnp.int32, (_D0_BLK, _D0_BLK), 1)
        eye = (r_i == c_i).astype(jnp.float32)
        t8 = lax.dot_general(w8, eye, (((0,), (0,)), ((), ())),
                             precision=lax.Precision.HIGHEST,
                             preferred_element_type=jnp.float32)
        o_ref[...] = jnp.broadcast_to(t8[:, :, None], (_F, _D0_BLK, _T))


def kernel(x, W0, W1, W2):
    del x
    w = jnp.stack([W0, W1, W2])
    out = pl.pallas_call(
        _body,
        grid=(3, _L // _D0_BLK),
        in_specs=[pl.BlockSpec((1, 50, _F), lambda i, j: (i, 0, 0))],
        out_specs=pl.BlockSpec((_F, _D0_BLK, _T), lambda i, j: (i, j, 0)),
        out_shape=jax.ShapeDtypeStruct((3 * _F, _L, _T), jnp.float32),
    )(w)
    return out.reshape(3 * _F, _L, _L, _L)
